# baseline (speedup 1.0000x reference)
import jax
import jax.numpy as jnp
from jax import lax
from jax.experimental import pallas as pl
from jax.experimental.pallas import tpu as pltpu
from jax.experimental.pallas import tpu_sc as plsc

D = 64
L = 16
CH = 128
W = 128
FMT_COLS = 512


def _tc_format(wt):
    V = wt.shape[1]
    grid = (V + FMT_COLS - 1) // FMT_COLS

    def body(in_ref, out_ref):
        out_ref[:, pl.ds(0, D)] = in_ref[...].T
        out_ref[:, pl.ds(D, W - D)] = jnp.zeros(
            (FMT_COLS, W - D), jnp.float32)

    return pl.pallas_call(
        body,
        grid=(grid,),
        in_specs=[pl.BlockSpec((D, FMT_COLS), lambda j: (0, j))],
        out_specs=pl.BlockSpec((FMT_COLS, W), lambda j: (j, 0)),
        out_shape=jax.ShapeDtypeStruct((V, W), jnp.float32),
    )(wt)


def _build(B, S, n_time, P):
    info = plsc.get_sparse_core_info()
    NC = info.num_cores
    ST = S // CH
    JOBS = 4
    mesh = plsc.VectorSubcoreMesh(core_axis_name="c", subcore_axis_name="s")

    def body(ids_i, ids_t, w2, w_pos, w_time, out_i, out_t, out_time,
             idx_v, rows_v, obuf, tbuf,
             gsem0, gsem1, gsem2, gsem3, psem0, psem1, psem2, psem3, osem):
        gsems = (gsem0, gsem1, gsem2, gsem3)
        psems = (psem0, psem1, psem2, psem3)
        wid = lax.axis_index("s") * NC + lax.axis_index("c")
        ct0 = lax.rem(wid * 2, ST)
        b = lax.div(wid * 2, ST)

        jobs = ((ids_i, out_i, 0), (ids_i, out_i, 1),
                (ids_t, out_t, 0), (ids_t, out_t, 1))

        pcopies = []
        gcopies = []
        for t, (ids4, _, j) in enumerate(jobs):
            pcopies.append(pltpu.async_copy(
                w_pos.at[:, ct0 + j], obuf.at[t], psems[t]))
            pltpu.sync_copy(ids4.at[ct0 + j, b], idx_v.at[t])
            gcopies.append(pltpu.async_copy(
                w2.at[idx_v.at[t]], rows_v.at[t], gsems[t]))

        iota = lax.iota(jnp.int32, L)
        perms = [lax.rem(iota + i, L) for i in range(L)]
        dts = [lax.div(p, 8) for p in perms]
        dss = [lax.rem(p, 8) for p in perms]

        ocopies = []
        for t, (_, out, j) in enumerate(jobs):
            pcopies[t].wait()
            gcopies[t].wait()
            rows_t = rows_v.at[t]
            obuf_t = obuf.at[t]

            @plsc.parallel_loop(0, CH // L, unroll=2)
            def _(r16, rows_t=rows_t, obuf_t=obuf_t):
                sl = iota + r16 * L
                for d0 in range(0, D, L):
                    for i in range(L):
                        x = plsc.load_gather(
                            rows_t, [sl, perms[i] + d0])
                        plsc.addupdate_scatter(
                            obuf_t, [dts[i] + (d0 // 8), dss[i], sl], x)
            ocopies.append(pltpu.async_copy(
                obuf_t, out.at[b, :, ct0 + j], osem))
        for cp in ocopies:
            cp.wait()

        @pl.when(wid == 0)
        def _():
            pltpu.sync_copy(w_time, tbuf)
            pltpu.sync_copy(tbuf, out_time.at[0])
            pltpu.sync_copy(tbuf, out_time.at[1])

    return pl.kernel(
        body,
        out_type=(
            jax.ShapeDtypeStruct((B, D // 8, ST, 8, CH), jnp.float32),
            jax.ShapeDtypeStruct((B, D // 8, ST, 8, CH), jnp.float32),
            jax.ShapeDtypeStruct((2, n_time, D), jnp.float32),
        ),
        mesh=mesh,
        compiler_params=pltpu.CompilerParams(use_tc_tiling_on_sc=False,
                                              needs_layout_passes=False),
        scratch_types=[
            pltpu.VMEM((JOBS, CH), jnp.int32),
            pltpu.VMEM((JOBS, CH, W), jnp.float32),
            pltpu.VMEM((JOBS, D // 8, 8, CH), jnp.float32),
            pltpu.VMEM((n_time, D), jnp.float32),
            pltpu.SemaphoreType.DMA,
            pltpu.SemaphoreType.DMA,
            pltpu.SemaphoreType.DMA,
            pltpu.SemaphoreType.DMA,
            pltpu.SemaphoreType.DMA,
            pltpu.SemaphoreType.DMA,
            pltpu.SemaphoreType.DMA,
            pltpu.SemaphoreType.DMA,
            pltpu.SemaphoreType.DMA,
        ],
    )


def kernel(input_ids, target_ids, W_word, W_pos, W_time):
    B, S = input_ids.shape
    n_time = W_time.shape[0]
    P = W_pos.shape[0]
    ST = S // CH
    ids_i = input_ids.astype(jnp.int32).reshape(B, ST, CH).transpose(1, 0, 2)
    ids_t = target_ids.astype(jnp.int32).reshape(B, ST, CH).transpose(1, 0, 2)
    pos5 = W_pos.reshape(P // CH, CH, D // 8, 8).transpose(2, 0, 3, 1)
    w2 = jnp.concatenate(
        [W_word, jnp.zeros((W_word.shape[0], W - D), jnp.float32)], axis=1)
    k = _build(B, S, n_time, P)
    out_i, out_t, out_time = k(ids_i, ids_t, w2, pos5, W_time)
    emb_i = out_i.transpose(0, 2, 4, 1, 3).reshape(B, S, D)
    emb_t = out_t.transpose(0, 2, 4, 1, 3).reshape(B, S, D)
    return (emb_i, emb_t, out_time.reshape(1, 2 * n_time, D))


# speedup vs baseline: 1.1289x; 1.1289x over previous
import jax
import jax.numpy as jnp
from jax import lax
from jax.experimental import pallas as pl
from jax.experimental.pallas import tpu as pltpu
from jax.experimental.pallas import tpu_sc as plsc

D = 64
L = 16
CH = 128
W = 128
FMT_COLS = 512


def _tc_format(wt):
    V = wt.shape[1]
    grid = (V + FMT_COLS - 1) // FMT_COLS

    def body(in_ref, out_ref):
        out_ref[:, pl.ds(0, D)] = in_ref[...].T
        out_ref[:, pl.ds(D, W - D)] = jnp.zeros(
            (FMT_COLS, W - D), jnp.float32)

    return pl.pallas_call(
        body,
        grid=(grid,),
        in_specs=[pl.BlockSpec((D, FMT_COLS), lambda j: (0, j))],
        out_specs=pl.BlockSpec((FMT_COLS, W), lambda j: (j, 0)),
        out_shape=jax.ShapeDtypeStruct((V, W), jnp.float32),
    )(wt)


def _build(B, S, n_time, P):
    info = plsc.get_sparse_core_info()
    NC = info.num_cores
    ST = S // CH
    JOBS = 4
    mesh = plsc.VectorSubcoreMesh(core_axis_name="c", subcore_axis_name="s")

    def body(ids_i, ids_t, w2, w_pos, w_time, out_i, out_t, out_time,
             idx_v, rows_v, obuf, tbuf,
             gsem0, gsem1, gsem2, gsem3, psem0, psem1, psem2, psem3, osem):
        gsems = (gsem0, gsem1, gsem2, gsem3)
        psems = (psem0, psem1, psem2, psem3)
        wid = lax.axis_index("s") * NC + lax.axis_index("c")
        ct0 = lax.rem(wid * 2, ST)
        b = lax.div(wid * 2, ST)

        jobs = ((ids_i, out_i, 0), (ids_i, out_i, 1),
                (ids_t, out_t, 0), (ids_t, out_t, 1))

        pcopies = []
        gcopies = []
        for t, (ids4, _, j) in enumerate(jobs):
            pcopies.append(pltpu.async_copy(
                w_pos.at[:, ct0 + j], obuf.at[t], psems[t]))
            pltpu.sync_copy(ids4.at[ct0 + j, b], idx_v.at[t])
            gcopies.append(pltpu.async_copy(
                w2.at[idx_v.at[t]], rows_v.at[t], gsems[t]))

        iota = lax.iota(jnp.int32, L)
        perms = [lax.rem(iota + i, L) for i in range(L)]
        dts = [lax.div(p, 8) for p in perms]
        dss = [lax.rem(p, 8) for p in perms]

        ocopies = []
        for t, (_, out, j) in enumerate(jobs):
            pcopies[t].wait()
            gcopies[t].wait()
            rows_t = rows_v.at[t]
            obuf_t = obuf.at[t]

            def blk_fn(r16, carry, rows_t=rows_t, obuf_t=obuf_t):
                sl = iota + r16 * L
                for d0 in range(0, D, L):
                    for i in range(L):
                        x = plsc.load_gather(
                            rows_t, [sl, perms[i] + d0])
                        plsc.addupdate_scatter(
                            obuf_t, [dts[i] + (d0 // 8), dss[i], sl], x)
                return carry

            lax.fori_loop(0, CH // L, blk_fn, 0, unroll=2)
            ocopies.append(pltpu.async_copy(
                obuf_t, out.at[b, :, ct0 + j], osem))
        for cp in ocopies:
            cp.wait()

        @pl.when(wid == 0)
        def _():
            pltpu.sync_copy(w_time, tbuf)
            pltpu.sync_copy(tbuf, out_time.at[0])
            pltpu.sync_copy(tbuf, out_time.at[1])

    return pl.kernel(
        body,
        out_type=(
            jax.ShapeDtypeStruct((B, D // 8, ST, 8, CH), jnp.float32),
            jax.ShapeDtypeStruct((B, D // 8, ST, 8, CH), jnp.float32),
            jax.ShapeDtypeStruct((2, n_time, D), jnp.float32),
        ),
        mesh=mesh,
        compiler_params=pltpu.CompilerParams(use_tc_tiling_on_sc=False,
                                              needs_layout_passes=False),
        scratch_types=[
            pltpu.VMEM((JOBS, CH), jnp.int32),
            pltpu.VMEM((JOBS, CH, W), jnp.float32),
            pltpu.VMEM((JOBS, D // 8, 8, CH), jnp.float32),
            pltpu.VMEM((n_time, D), jnp.float32),
            pltpu.SemaphoreType.DMA,
            pltpu.SemaphoreType.DMA,
            pltpu.SemaphoreType.DMA,
            pltpu.SemaphoreType.DMA,
            pltpu.SemaphoreType.DMA,
            pltpu.SemaphoreType.DMA,
            pltpu.SemaphoreType.DMA,
            pltpu.SemaphoreType.DMA,
            pltpu.SemaphoreType.DMA,
        ],
    )


def kernel(input_ids, target_ids, W_word, W_pos, W_time):
    B, S = input_ids.shape
    n_time = W_time.shape[0]
    P = W_pos.shape[0]
    ST = S // CH
    ids_i = input_ids.astype(jnp.int32).reshape(B, ST, CH).transpose(1, 0, 2)
    ids_t = target_ids.astype(jnp.int32).reshape(B, ST, CH).transpose(1, 0, 2)
    pos5 = W_pos.reshape(P // CH, CH, D // 8, 8).transpose(2, 0, 3, 1)
    w2 = jnp.concatenate(
        [W_word, jnp.zeros((W_word.shape[0], W - D), jnp.float32)], axis=1)
    k = _build(B, S, n_time, P)
    out_i, out_t, out_time = k(ids_i, ids_t, w2, pos5, W_time)
    emb_i = out_i.transpose(0, 2, 4, 1, 3).reshape(B, S, D)
    emb_t = out_t.transpose(0, 2, 4, 1, 3).reshape(B, S, D)
    return (emb_i, emb_t, out_time.reshape(1, 2 * n_time, D))
